# baseline (device time: 186130 ns/iter reference)
import jax
import jax.numpy as jnp
import numpy as np
from jax import lax
from jax.experimental import pallas as pl
from jax.experimental.pallas import tpu as pltpu

N_DEV = 32
N_STREAMS = 8
S2 = N_STREAMS // 2


def _ring_tables():
    yz = [(0, 0), (0, 1), (0, 2), (0, 3), (1, 3), (1, 2), (1, 1), (2, 1),
          (2, 2), (2, 3), (3, 3), (3, 2), (3, 1), (3, 0), (2, 0), (1, 0)]
    plane = {(0, 0): 0, (1, 0): 1, (1, 1): 2, (0, 1): 3,
             (0, 2): 4, (1, 2): 5, (1, 3): 6, (0, 3): 7}
    coords = []
    for i, (y, z) in enumerate(yz):
        for x in ((0, 1) if i % 2 == 0 else (1, 0)):
            coords.append((x, y, z))
    assert all(
        sum(abs(u - v) for u, v in zip(coords[r], coords[(r + 1) % 32])) == 1
        for r in range(32)
    )
    perm = [z * 8 + plane[(x, y)] for (x, y, z) in coords]
    inv = [0] * N_DEV
    for r, m in enumerate(perm):
        inv[m] = r
    return np.array(perm, np.int32), np.array(inv, np.int32)


_PERM, _INV = _ring_tables()


def kernel(x, w_mat, scale_x, scale_w):
    m_global, k_sh = x.shape
    _, n = w_mat.shape
    m_blk = m_global // N_DEV
    half = n // 2
    sub = half // S2
    s2 = (scale_x * scale_w).reshape(1, 1)

    ORDER = [k + d * S2 for k in range(S2) for d in (0, 1)]

    def body(x_ref, w_ref, s_ref, perm_ref, inv_ref, out_ref,
             send_ref, recv_ref, send_sems, recv_sems):
        my = lax.axis_index("i")
        rp = inv_ref[my]
        right = perm_ref[lax.rem(rp + 1, N_DEV)]
        left = perm_ref[lax.rem(rp + N_DEV - 1, N_DEV)]

        barrier_sem = pltpu.get_barrier_semaphore()
        for nbr in (left, right):
            pl.semaphore_signal(
                barrier_sem, inc=1,
                device_id=(nbr,), device_id_type=pl.DeviceIdType.MESH,
            )
        pl.semaphore_wait(barrier_sem, 2)

        def rdma(s, h, tgt):
            return pltpu.make_async_remote_copy(
                src_ref=send_ref.at[s, h % 2],
                dst_ref=recv_ref.at[s, h],
                send_sem=send_sems.at[s, h % 2],
                recv_sem=recv_sems.at[s, h],
                device_id=(tgt,),
                device_id_type=pl.DeviceIdType.MESH,
            )

        def partial(c, col0, ncol):
            xb = x_ref[pl.ds(c * m_blk, m_blk), :]
            return lax.dot_general(
                xb, w_ref[:, col0:col0 + ncol],
                (((1,), (0,)), ((), ())),
                preferred_element_type=jnp.int32,
            )

        for h in range(N_DEV - 1):
            c_cw = perm_ref[lax.rem(rp + N_DEV - h - 1, N_DEV)]
            c_ccw = perm_ref[lax.rem(rp + h + 1, N_DEV)]
            p_cw = partial(c_cw, 0, half)
            p_ccw = partial(c_ccw, half, half)
            for s in ORDER:
                cw = s < S2
                tgt = right if cw else left
                p = p_cw if cw else p_ccw
                c0 = (s % S2) * sub
                if h >= 2:
                    rdma(s, h - 2, tgt).wait_send()
                if h == 0:
                    val = p[:, c0:c0 + sub]
                else:
                    rdma(s, h - 1, tgt).wait_recv()
                    val = recv_ref[s, h - 1] + p[:, c0:c0 + sub]
                send_ref[s, h % 2] = val
                rdma(s, h, tgt).start()

        pm = partial(my, 0, n)
        scale = s_ref[0, 0]
        for s in ORDER:
            cw = s < S2
            tgt = right if cw else left
            g0 = (0 if cw else half) + (s % S2) * sub
            rdma(s, N_DEV - 2, tgt).wait_recv()
            acc = recv_ref[s, N_DEV - 2] + pm[:, g0:g0 + sub]
            out_ref[:, g0:g0 + sub] = jnp.maximum(
                acc.astype(jnp.float32) * scale, 0.0
            )
        for s in range(N_STREAMS):
            tgt = right if s < S2 else left
            rdma(s, N_DEV - 3, tgt).wait_send()
            rdma(s, N_DEV - 2, tgt).wait_send()

    return pl.pallas_call(
        body,
        out_shape=jax.ShapeDtypeStruct((m_blk, n), jnp.float32),
        in_specs=[
            pl.BlockSpec(memory_space=pltpu.VMEM),
            pl.BlockSpec(memory_space=pltpu.VMEM),
            pl.BlockSpec(memory_space=pltpu.SMEM),
            pl.BlockSpec(memory_space=pltpu.SMEM),
            pl.BlockSpec(memory_space=pltpu.SMEM),
        ],
        out_specs=pl.BlockSpec(memory_space=pltpu.VMEM),
        scratch_shapes=[
            pltpu.VMEM((N_STREAMS, 2, m_blk, sub), jnp.int32),
            pltpu.VMEM((N_STREAMS, N_DEV - 1, m_blk, sub), jnp.int32),
            pltpu.SemaphoreType.DMA((N_STREAMS, 2)),
            pltpu.SemaphoreType.DMA((N_STREAMS, N_DEV - 1)),
        ],
        compiler_params=pltpu.CompilerParams(
            collective_id=0, vmem_limit_bytes=64 * 1024 * 1024
        ),
    )(x, w_mat, s2, jnp.asarray(_PERM), jnp.asarray(_INV))


# device time: 99079 ns/iter; 1.8786x vs baseline; 1.8786x over previous
import jax
import jax.numpy as jnp
import numpy as np
from jax import lax
from jax.experimental import pallas as pl
from jax.experimental.pallas import tpu as pltpu

N_DEV = 32
N_STREAMS = 8
S2 = N_STREAMS // 2


def _ring_tables():
    yz = [(0, 0), (0, 1), (0, 2), (0, 3), (1, 3), (1, 2), (1, 1), (2, 1),
          (2, 2), (2, 3), (3, 3), (3, 2), (3, 1), (3, 0), (2, 0), (1, 0)]
    plane = {(0, 0): 0, (1, 0): 1, (1, 1): 2, (0, 1): 3,
             (0, 2): 4, (1, 2): 5, (1, 3): 6, (0, 3): 7}
    coords = []
    for i, (y, z) in enumerate(yz):
        for x in ((0, 1) if i % 2 == 0 else (1, 0)):
            coords.append((x, y, z))
    assert all(
        sum(abs(u - v) for u, v in zip(coords[r], coords[(r + 1) % 32])) == 1
        for r in range(32)
    )
    perm = [z * 8 + plane[(x, y)] for (x, y, z) in coords]
    inv = [0] * N_DEV
    for r, m in enumerate(perm):
        inv[m] = r
    return np.array(perm, np.int32), np.array(inv, np.int32)


_PERM, _INV = _ring_tables()

_Q = [(h + 1) * 2064512 / 32000.0 for h in range(N_DEV - 1)]


def kernel(x, w_mat, scale_x, scale_w):
    m_global, k_sh = x.shape
    _, n = w_mat.shape
    m_blk = m_global // N_DEV
    half = n // 2
    sub = half // S2
    s2 = (scale_x * scale_w).reshape(1, 1)

    ORDER = [k + d * S2 for k in range(S2) for d in (0, 1)]

    def body(x_ref, w_ref, s_ref, perm_ref, inv_ref, out_ref,
             send_ref, recv_ref, send_sems, recv_sems):
        my = lax.axis_index("i")
        rp = inv_ref[my]
        right = perm_ref[lax.rem(rp + 1, N_DEV)]
        left = perm_ref[lax.rem(rp + N_DEV - 1, N_DEV)]

        barrier_sem = pltpu.get_barrier_semaphore()
        for nbr in (left, right):
            pl.semaphore_signal(
                barrier_sem, inc=1,
                device_id=(nbr,), device_id_type=pl.DeviceIdType.MESH,
            )
        pl.semaphore_wait(barrier_sem, 2)

        def rdma(s, h, tgt):
            return pltpu.make_async_remote_copy(
                src_ref=send_ref.at[s, h % 2],
                dst_ref=recv_ref.at[s, h],
                send_sem=send_sems.at[s, h % 2],
                recv_sem=recv_sems.at[s, h],
                device_id=(tgt,),
                device_id_type=pl.DeviceIdType.MESH,
            )

        def partial(c, col0, ncol):
            xb = x_ref[pl.ds(c * m_blk, m_blk), :]
            return lax.dot_general(
                xb, w_ref[:, col0:col0 + ncol],
                (((1,), (0,)), ((), ())),
                preferred_element_type=jnp.int32,
            )

        for h in range(N_DEV - 1):
            c_cw = perm_ref[lax.rem(rp + N_DEV - h - 1, N_DEV)]
            c_ccw = perm_ref[lax.rem(rp + h + 1, N_DEV)]
            p_cw = partial(c_cw, 0, half)
            p_ccw = partial(c_ccw, half, half)
            for s in ORDER:
                cw = s < S2
                tgt = right if cw else left
                p = p_cw if cw else p_ccw
                c0 = (s % S2) * sub
                if h >= 2:
                    rdma(s, h - 2, tgt).wait_send()
                if h == 0:
                    val = p[:, c0:c0 + sub].astype(jnp.float32)
                else:
                    rdma(s, h - 1, tgt).wait_recv()
                    val = (
                        recv_ref[s, h - 1].astype(jnp.float32) * _Q[h - 1]
                        + p[:, c0:c0 + sub].astype(jnp.float32)
                    )
                send_ref[s, h % 2] = jnp.clip(
                    jnp.rint(val * (1.0 / _Q[h])), -32767.0, 32767.0
                ).astype(jnp.int16)
                rdma(s, h, tgt).start()

        pm = partial(my, 0, n)
        scale = s_ref[0, 0]
        for s in ORDER:
            cw = s < S2
            tgt = right if cw else left
            g0 = (0 if cw else half) + (s % S2) * sub
            rdma(s, N_DEV - 2, tgt).wait_recv()
            acc = (
                recv_ref[s, N_DEV - 2].astype(jnp.float32) * _Q[N_DEV - 2]
                + pm[:, g0:g0 + sub].astype(jnp.float32)
            )
            out_ref[:, g0:g0 + sub] = jnp.maximum(acc * scale, 0.0)
        for s in range(N_STREAMS):
            tgt = right if s < S2 else left
            rdma(s, N_DEV - 3, tgt).wait_send()
            rdma(s, N_DEV - 2, tgt).wait_send()

    return pl.pallas_call(
        body,
        out_shape=jax.ShapeDtypeStruct((m_blk, n), jnp.float32),
        in_specs=[
            pl.BlockSpec(memory_space=pltpu.VMEM),
            pl.BlockSpec(memory_space=pltpu.VMEM),
            pl.BlockSpec(memory_space=pltpu.SMEM),
            pl.BlockSpec(memory_space=pltpu.SMEM),
            pl.BlockSpec(memory_space=pltpu.SMEM),
        ],
        out_specs=pl.BlockSpec(memory_space=pltpu.VMEM),
        scratch_shapes=[
            pltpu.VMEM((N_STREAMS, 2, m_blk, sub), jnp.int16),
            pltpu.VMEM((N_STREAMS, N_DEV - 1, m_blk, sub), jnp.int16),
            pltpu.SemaphoreType.DMA((N_STREAMS, 2)),
            pltpu.SemaphoreType.DMA((N_STREAMS, N_DEV - 1)),
        ],
        compiler_params=pltpu.CompilerParams(
            collective_id=0, vmem_limit_bytes=64 * 1024 * 1024
        ),
    )(x, w_mat, s2, jnp.asarray(_PERM), jnp.asarray(_INV))


# device time: 71895 ns/iter; 2.5889x vs baseline; 1.3781x over previous
import jax
import jax.numpy as jnp
import numpy as np
from jax import lax
from jax.experimental import pallas as pl
from jax.experimental.pallas import tpu as pltpu

N_DEV = 32
CW_HOPS = 16
CCW_HOPS = 15
SUBS = 4


def _ring_tables():
    yz = [(0, 0), (0, 1), (0, 2), (0, 3), (1, 3), (1, 2), (1, 1), (2, 1),
          (2, 2), (2, 3), (3, 3), (3, 2), (3, 1), (3, 0), (2, 0), (1, 0)]
    plane = {(0, 0): 0, (1, 0): 1, (1, 1): 2, (0, 1): 3,
             (0, 2): 4, (1, 2): 5, (1, 3): 6, (0, 3): 7}
    coords = []
    for i, (y, z) in enumerate(yz):
        for x in ((0, 1) if i % 2 == 0 else (1, 0)):
            coords.append((x, y, z))
    assert all(
        sum(abs(u - v) for u, v in zip(coords[r], coords[(r + 1) % 32])) == 1
        for r in range(32)
    )
    perm = [z * 8 + plane[(x, y)] for (x, y, z) in coords]
    inv = [0] * N_DEV
    for r, m in enumerate(perm):
        inv[m] = r
    return np.array(perm, np.int32), np.array(inv, np.int32)


_PERM, _INV = _ring_tables()


def kernel(x, w_mat, scale_x, scale_w):
    m_global, k_sh = x.shape
    _, n = w_mat.shape
    m_blk = m_global // N_DEV
    sub = n // SUBS
    s2 = (scale_x * scale_w).reshape(1, 1)

    def body(x_ref, w_ref, s_ref, perm_ref, inv_ref, out_ref,
             xg_ref, wg_cw, wg_ccw,
             x_send_sems, x_recv_sems,
             cw_send_sems, cw_recv_sems, ccw_send_sems, ccw_recv_sems):
        my = lax.axis_index("i")
        rp = inv_ref[my]
        right = perm_ref[lax.rem(rp + 1, N_DEV)]
        left = perm_ref[lax.rem(rp + N_DEV - 1, N_DEV)]

        barrier_sem = pltpu.get_barrier_semaphore()
        for o in range(1, N_DEV):
            pl.semaphore_signal(
                barrier_sem, inc=1,
                device_id=(lax.rem(my + o, N_DEV),),
                device_id_type=pl.DeviceIdType.MESH,
            )
        pl.semaphore_wait(barrier_sem, N_DEV - 1)

        def x_rdma(o):
            d = lax.rem(my + o, N_DEV)
            return pltpu.make_async_remote_copy(
                src_ref=x_ref.at[pl.ds(d * m_blk, m_blk), :],
                dst_ref=xg_ref.at[my],
                send_sem=x_send_sems.at[o - 1],
                recv_sem=x_recv_sems.at[my],
                device_id=(d,),
                device_id_type=pl.DeviceIdType.MESH,
            )

        def x_recv(src):
            return pltpu.make_async_remote_copy(
                src_ref=x_ref.at[pl.ds(0, m_blk), :],
                dst_ref=xg_ref.at[src],
                send_sem=x_send_sems.at[0],
                recv_sem=x_recv_sems.at[src],
                device_id=(src,),
                device_id_type=pl.DeviceIdType.MESH,
            )

        def w_rdma(t, h, cw):
            buf, ssem, rsem, tgt = (
                (wg_cw, cw_send_sems, cw_recv_sems, right) if cw
                else (wg_ccw, ccw_send_sems, ccw_recv_sems, left)
            )
            src = (
                w_ref.at[:, pl.ds(t * sub, sub)] if h == 0
                else buf.at[t, h - 1]
            )
            return pltpu.make_async_remote_copy(
                src_ref=src,
                dst_ref=buf.at[t, h],
                send_sem=ssem.at[t, h],
                recv_sem=rsem.at[t, h],
                device_id=(tgt,),
                device_id_type=pl.DeviceIdType.MESH,
            )

        for o in range(1, N_DEV):
            x_rdma(o).start()
        for t in range(SUBS):
            w_rdma(t, 0, True).start()
            w_rdma(t, 0, False).start()
        for o in range(1, N_DEV):
            x_recv(lax.rem(my + o, N_DEV)).wait_recv()

        xb_my = x_ref[pl.ds(my * m_blk, m_blk), :]
        acc = [
            lax.dot_general(
                xb_my, w_ref[:, t * sub:(t + 1) * sub],
                (((1,), (0,)), ((), ())),
                preferred_element_type=jnp.int32,
            )
            for t in range(SUBS)
        ]

        def gemm_in(origin, buf, t, h, a):
            xb = xg_ref[origin]
            return a + lax.dot_general(
                xb, buf[t, h], (((1,), (0,)), ((), ())),
                preferred_element_type=jnp.int32,
            )

        for h in range(CW_HOPS):
            for t in range(SUBS):
                w_rdma(t, h, True).wait_recv()
                if h + 1 < CW_HOPS:
                    w_rdma(t, h + 1, True).start()
            if h < CCW_HOPS:
                for t in range(SUBS):
                    w_rdma(t, h, False).wait_recv()
                    if h + 1 < CCW_HOPS:
                        w_rdma(t, h + 1, False).start()
            o_cw = perm_ref[lax.rem(rp + N_DEV - 1 - h, N_DEV)]
            for t in range(SUBS):
                acc[t] = gemm_in(o_cw, wg_cw, t, h, acc[t])
            if h < CCW_HOPS:
                o_ccw = perm_ref[lax.rem(rp + 1 + h, N_DEV)]
                for t in range(SUBS):
                    acc[t] = gemm_in(o_ccw, wg_ccw, t, h, acc[t])

        scale = s_ref[0, 0]
        for t in range(SUBS):
            out_ref[:, t * sub:(t + 1) * sub] = jnp.maximum(
                acc[t].astype(jnp.float32) * scale, 0.0
            )

        for o in range(1, N_DEV):
            x_rdma(o).wait_send()
        for t in range(SUBS):
            for h in range(CW_HOPS):
                w_rdma(t, h, True).wait_send()
            for h in range(CCW_HOPS):
                w_rdma(t, h, False).wait_send()

    return pl.pallas_call(
        body,
        out_shape=jax.ShapeDtypeStruct((m_blk, n), jnp.float32),
        in_specs=[
            pl.BlockSpec(memory_space=pltpu.VMEM),
            pl.BlockSpec(memory_space=pltpu.VMEM),
            pl.BlockSpec(memory_space=pltpu.SMEM),
            pl.BlockSpec(memory_space=pltpu.SMEM),
            pl.BlockSpec(memory_space=pltpu.SMEM),
        ],
        out_specs=pl.BlockSpec(memory_space=pltpu.VMEM),
        scratch_shapes=[
            pltpu.VMEM((N_DEV, m_blk, k_sh), jnp.int8),
            pltpu.VMEM((SUBS, CW_HOPS, k_sh, sub), jnp.int8),
            pltpu.VMEM((SUBS, CCW_HOPS, k_sh, sub), jnp.int8),
            pltpu.SemaphoreType.DMA((N_DEV - 1,)),
            pltpu.SemaphoreType.DMA((N_DEV,)),
            pltpu.SemaphoreType.DMA((SUBS, CW_HOPS)),
            pltpu.SemaphoreType.DMA((SUBS, CW_HOPS)),
            pltpu.SemaphoreType.DMA((SUBS, CCW_HOPS)),
            pltpu.SemaphoreType.DMA((SUBS, CCW_HOPS)),
        ],
        compiler_params=pltpu.CompilerParams(
            collective_id=0, vmem_limit_bytes=64 * 1024 * 1024
        ),
    )(x, w_mat, s2, jnp.asarray(_PERM), jnp.asarray(_INV))
